# baseline (speedup 1.0000x reference)
import functools

import jax
import jax.numpy as jnp
from jax import lax
from jax.experimental import pallas as pl
from jax.experimental.pallas import tpu as pltpu
from jax.experimental.pallas import tpu_sc as plsc

D_MODEL = 64
GATHER_W = 128
K = 8
NC, NS = 2, 16
NW = NC * NS

_mesh = plsc.VectorSubcoreMesh(core_axis_name="c", subcore_axis_name="s")


def _make_gather(n_blocks: int):
    assert n_blocks % NW == 0
    blocks_per_w = n_blocks // NW
    assert blocks_per_w % K == 0
    n_chunks = blocks_per_w // K

    @functools.partial(
        pl.kernel,
        mesh=_mesh,
        compiler_params=pltpu.CompilerParams(use_tc_tiling_on_sc=False),
        out_type=jax.ShapeDtypeStruct((n_blocks, GATHER_W, D_MODEL), jnp.float32),
        scratch_types=[
            pltpu.VMEM((K, GATHER_W), jnp.int32),
            pltpu.VMEM((K, GATHER_W, D_MODEL), jnp.float32),
            pltpu.SemaphoreType.DMA,
        ],
    )
    def _gather(idx_hbm, table_hbm, out_hbm, idx_v, rows_v, sem):
        wid = lax.axis_index("s") * NC + lax.axis_index("c")
        base = wid * blocks_per_w

        def chunk(c, carry):
            row0 = base + c * K
            pltpu.sync_copy(idx_hbm.at[pl.ds(row0, K)], idx_v)
            copies = [
                pltpu.async_copy(table_hbm.at[idx_v.at[j]], rows_v.at[j], sem)
                for j in range(K)
            ]
            for cp in copies:
                cp.wait()
            pltpu.sync_copy(rows_v, out_hbm.at[pl.ds(row0, K)])
            return carry

        lax.fori_loop(0, n_chunks, chunk, 0)

    return _gather


def kernel(x, table):
    b, h = x.shape
    flat = x.reshape(-1).astype(jnp.int32)
    n_blocks = flat.shape[0] // GATHER_W
    idx2d = flat.reshape(n_blocks, GATHER_W)
    out = _make_gather(n_blocks)(idx2d, table)
    return out.reshape(b, h, table.shape[1])


# speedup vs baseline: 1.0185x; 1.0185x over previous
import functools

import jax
import jax.numpy as jnp
from jax import lax
from jax.experimental import pallas as pl
from jax.experimental.pallas import tpu as pltpu
from jax.experimental.pallas import tpu_sc as plsc

D_MODEL = 64
GATHER_W = 128
K = 5
NBUF = 2
NC, NS = 2, 16
NW = NC * NS

_mesh = plsc.VectorSubcoreMesh(core_axis_name="c", subcore_axis_name="s")


def _make_gather(n_blocks: int):
    assert n_blocks % NW == 0
    blocks_per_w = n_blocks // NW
    assert blocks_per_w % K == 0
    n_chunks = blocks_per_w // K
    assert n_chunks % NBUF == 0

    @functools.partial(
        pl.kernel,
        mesh=_mesh,
        compiler_params=pltpu.CompilerParams(use_tc_tiling_on_sc=False),
        out_type=jax.ShapeDtypeStruct((n_blocks, GATHER_W, D_MODEL), jnp.float32),
        scratch_types=[
            pltpu.VMEM((blocks_per_w, GATHER_W), jnp.int32),
            pltpu.VMEM((NBUF, K, GATHER_W, D_MODEL), jnp.float32),
            pltpu.SemaphoreType.DMA,
            pltpu.SemaphoreType.DMA,
        ],
    )
    def _gather(idx_hbm, table_hbm, out_hbm, idx_v, rows_v, sem0, sem1):
        wid = lax.axis_index("s") * NC + lax.axis_index("c")
        base = wid * blocks_per_w
        sems = [sem0, sem1]

        pltpu.sync_copy(idx_hbm.at[pl.ds(base, blocks_per_w)], idx_v)

        def fire(c, buf):
            for j in range(K):
                pltpu.async_copy(
                    table_hbm.at[idx_v.at[c * K + j]], rows_v.at[buf].at[j],
                    sems[buf],
                )

        def drain(buf):
            pltpu.make_async_copy(
                out_hbm.at[pl.ds(0, K)], rows_v.at[buf], sems[buf]
            ).wait()

        fire(0, 0)

        def pair(p, carry):
            c0 = p * NBUF
            for b in range(NBUF):
                c = c0 + b
                nxt = (b + 1) % NBUF

                @pl.when(c + 1 < n_chunks)
                def _():
                    fire(c + 1, nxt)

                drain(b)
                pltpu.sync_copy(rows_v.at[b], out_hbm.at[pl.ds(base + c * K, K)])
            return carry

        lax.fori_loop(0, n_chunks // NBUF, pair, 0)

    return _gather


def kernel(x, table):
    b, h = x.shape
    flat = x.reshape(-1).astype(jnp.int32)
    n_blocks = flat.shape[0] // GATHER_W
    idx2d = flat.reshape(n_blocks, GATHER_W)
    out = _make_gather(n_blocks)(idx2d, table)
    return out.reshape(b, h, table.shape[1])
